# Initial kernel scaffold; baseline (speedup 1.0000x reference)
import functools

import jax
import jax.numpy as jnp
from jax import lax
from jax.experimental import pallas as pl
from jax.experimental.pallas import tpu as pltpu
from jax.experimental.pallas import tpu_sc as plsc

N_COMP = 1024
N_BG = 2
BATCH = 16
SCALE = 1.0 / 128.0

NC = 2
NS = 16
NW = NC * NS
G = 128


def _sc_body(xt_hbm, rows_hbm, cols_hbm, vals_hbm, out_hbm,
             idx_v, cols_v, vals_v, xr_v, ystage_v, yshared, sem,
             *, chunks_per_worker):
    cid = lax.axis_index("c")
    sid = lax.axis_index("s")
    wid = sid * NC + cid

    @pl.when(sid == 0)
    def _zero():
        def zrow(i, carry):
            ystage_v[i, :] = jnp.zeros((BATCH,), jnp.float32)
            return carry
        lax.fori_loop(0, N_COMP, zrow, 0)
        pltpu.sync_copy(ystage_v, yshared)

    plsc.subcore_barrier()

    base0 = wid * (chunks_per_worker * G)

    def chunk_body(t, carry):
        base = base0 + t * G
        pltpu.sync_copy(rows_hbm.at[pl.ds(base, G)], idx_v)
        pltpu.sync_copy(cols_hbm.at[pl.ds(base, G)], cols_v)
        pltpu.sync_copy(vals_hbm.at[pl.ds(base, G)], vals_v)
        pltpu.async_copy(xt_hbm.at[idx_v], xr_v, sem).wait()

        def row_body(j, c2):
            v = vals_v[j]
            xr_v[j, :] = xr_v[j, :] * v
            return c2
        lax.fori_loop(0, G, row_body, 0)

        pltpu.sync_copy(xr_v, yshared.at[cols_v], add=True)
        return carry

    lax.fori_loop(0, chunks_per_worker, chunk_body, 0)

    plsc.subcore_barrier()

    @pl.when(sid == 0)
    def _writeout():
        pltpu.sync_copy(yshared, ystage_v)
        pltpu.sync_copy(ystage_v, out_hbm.at[cid])


def _sc_partials(xt, rows, cols, vals, chunks_per_worker):
    mesh = plsc.VectorSubcoreMesh(core_axis_name="c", subcore_axis_name="s")
    body = functools.partial(_sc_body, chunks_per_worker=chunks_per_worker)
    return pl.kernel(
        body,
        out_type=jax.ShapeDtypeStruct((NC, N_COMP, BATCH), jnp.float32),
        mesh=mesh,
        scratch_types=[
            pltpu.VMEM((G,), jnp.int32),
            pltpu.VMEM((G,), jnp.int32),
            pltpu.VMEM((G,), jnp.float32),
            pltpu.VMEM((G, BATCH), jnp.float32),
            pltpu.VMEM((N_COMP, BATCH), jnp.float32),
            pltpu.VMEM_SHARED((N_COMP, BATCH), jnp.float32),
            pltpu.SemaphoreType.DMA,
        ],
    )(xt, rows, cols, vals)


def _tc_body(p_ref, x_ref, b_ref, o_ref):
    ysum = (p_ref[0] + p_ref[1]) * SCALE
    xb = jnp.dot(x_ref[...], b_ref[...], preferred_element_type=jnp.float32)
    y1 = xb.T * SCALE
    pad = jnp.zeros((6, BATCH), jnp.float32)
    o_ref[...] = jnp.concatenate([ysum, y1, pad], axis=0)


def _tc_merge(partials, X, b):
    return pl.pallas_call(
        _tc_body,
        out_shape=jax.ShapeDtypeStruct((N_COMP + N_BG + 6, BATCH), jnp.float32),
    )(partials, X, b)


def kernel(X, A_rows, A_cols, A_vals, b):
    nnz = A_rows.shape[0]
    per_round = NW * G
    nnz_pad = ((nnz + per_round - 1) // per_round) * per_round
    pad = nnz_pad - nnz
    rows = jnp.concatenate([A_rows, jnp.zeros((pad,), A_rows.dtype)])
    cols = jnp.concatenate([A_cols, jnp.zeros((pad,), A_cols.dtype)])
    vals = jnp.concatenate([A_vals, jnp.zeros((pad,), A_vals.dtype)])
    xt = X.T
    partials = _sc_partials(xt, rows, cols, vals, nnz_pad // per_round)
    out = _tc_merge(partials, X, b)
    return out[:N_COMP + N_BG]


# speedup vs baseline: 10.4629x; 10.4629x over previous
import functools

import jax
import jax.numpy as jnp
from jax import lax
from jax.experimental import pallas as pl
from jax.experimental.pallas import tpu as pltpu
from jax.experimental.pallas import tpu_sc as plsc

N_COMP = 1024
N_BG = 2
BATCH = 16
SCALE = 1.0 / 128.0

NC = 2
NS = 16
NW = NC * NS
G = 128


def _sc_body(xt_hbm, rows_hbm, cols_hbm, vals_hbm, out_hbm,
             idx_v, cols_v, vals_v, xr_v, ystage_v, yshared, sem,
             *, chunks_per_worker):
    cid = lax.axis_index("c")
    sid = lax.axis_index("s")
    wid = sid * NC + cid

    @pl.when(sid == 0)
    def _zero():
        def zrow(i, carry):
            ystage_v[i, :] = jnp.zeros((BATCH,), jnp.float32)
            return carry
        lax.fori_loop(0, N_COMP, zrow, 0)
        pltpu.sync_copy(ystage_v, yshared)

    plsc.subcore_barrier()

    base0 = wid * (chunks_per_worker * G)

    def chunk_body(t, carry):
        base = base0 + t * G
        pltpu.sync_copy(rows_hbm.at[pl.ds(base, G)], idx_v)
        pltpu.sync_copy(cols_hbm.at[pl.ds(base, G)], cols_v)
        pltpu.sync_copy(vals_hbm.at[pl.ds(base, G)], vals_v)
        pltpu.async_copy(xt_hbm.at[idx_v], xr_v, sem).wait()

        def grp_body(g, c2):
            vals16 = vals_v[pl.ds(g * 16, 16)]
            rbase = g * 16
            for jj in range(16):
                v = vals16[jj]
                xr_v[rbase + jj, :] = xr_v[rbase + jj, :] * v
            return c2
        lax.fori_loop(0, G // 16, grp_body, 0)

        pltpu.sync_copy(xr_v, yshared.at[cols_v], add=True)
        return carry

    lax.fori_loop(0, chunks_per_worker, chunk_body, 0)

    plsc.subcore_barrier()

    @pl.when(sid == 0)
    def _writeout():
        pltpu.sync_copy(yshared, ystage_v)
        pltpu.sync_copy(ystage_v, out_hbm.at[cid])


def _sc_partials(xt, rows, cols, vals, chunks_per_worker):
    mesh = plsc.VectorSubcoreMesh(core_axis_name="c", subcore_axis_name="s")
    body = functools.partial(_sc_body, chunks_per_worker=chunks_per_worker)
    return pl.kernel(
        body,
        out_type=jax.ShapeDtypeStruct((NC, N_COMP, BATCH), jnp.float32),
        mesh=mesh,
        scratch_types=[
            pltpu.VMEM((G,), jnp.int32),
            pltpu.VMEM((G,), jnp.int32),
            pltpu.VMEM((G,), jnp.float32),
            pltpu.VMEM((G, BATCH), jnp.float32),
            pltpu.VMEM((N_COMP, BATCH), jnp.float32),
            pltpu.VMEM_SHARED((N_COMP, BATCH), jnp.float32),
            pltpu.SemaphoreType.DMA,
        ],
        compiler_params=pltpu.CompilerParams(use_tc_tiling_on_sc=False),
    )(xt, rows, cols, vals)


def _tc_body(p_ref, x_ref, b_ref, o_ref):
    ysum = (p_ref[0] + p_ref[1]) * SCALE
    xb = jnp.dot(x_ref[...], b_ref[...], preferred_element_type=jnp.float32)
    y1 = xb.T * SCALE
    pad = jnp.zeros((6, BATCH), jnp.float32)
    o_ref[...] = jnp.concatenate([ysum, y1, pad], axis=0)


def _tc_merge(partials, X, b):
    return pl.pallas_call(
        _tc_body,
        out_shape=jax.ShapeDtypeStruct((N_COMP + N_BG + 6, BATCH), jnp.float32),
    )(partials, X, b)


def kernel(X, A_rows, A_cols, A_vals, b):
    nnz = A_rows.shape[0]
    per_round = NW * G
    nnz_pad = ((nnz + per_round - 1) // per_round) * per_round
    pad = nnz_pad - nnz
    rows = jnp.concatenate([A_rows, jnp.zeros((pad,), A_rows.dtype)])
    cols = jnp.concatenate([A_cols, jnp.zeros((pad,), A_cols.dtype)])
    vals = jnp.concatenate([A_vals, jnp.zeros((pad,), A_vals.dtype)])
    xt = X.T
    partials = _sc_partials(xt, rows, cols, vals, nnz_pad // per_round)
    out = _tc_merge(partials, X, b)
    return out[:N_COMP + N_BG]


# speedup vs baseline: 19.3431x; 1.8487x over previous
import functools

import jax
import jax.numpy as jnp
from jax import lax
from jax.experimental import pallas as pl
from jax.experimental.pallas import tpu as pltpu
from jax.experimental.pallas import tpu_sc as plsc

N_COMP = 1024
N_BG = 2
BATCH = 16
SCALE = 1.0 / 128.0

NC = 2
NS = 16
NW = NC * NS
G = 128
NBUF = 4


def _sc_body(xt_hbm, rows_hbm, cols_hbm, vals_hbm, out_hbm,
             rows_buf, cols_buf, vals_buf,
             xr0, xr1, xr2, xr3, ystage_v, yshared,
             g0, g1, g2, g3, s0, s1, s2, s3,
             *, nch):
    xr = (xr0, xr1, xr2, xr3)
    gs = (g0, g1, g2, g3)
    ss = (s0, s1, s2, s3)
    cid = lax.axis_index("c")
    sid = lax.axis_index("s")
    wid = sid * NC + cid

    @pl.when(sid == 0)
    def _zero():
        def zrow(i, carry):
            ystage_v[i, :] = jnp.zeros((BATCH,), jnp.float32)
            return carry
        lax.fori_loop(0, N_COMP, zrow, 0)
        pltpu.sync_copy(ystage_v, yshared)

    plsc.subcore_barrier()

    cbase = wid * nch
    pltpu.sync_copy(rows_hbm.at[pl.ds(cbase, nch)], rows_buf)
    pltpu.sync_copy(cols_hbm.at[pl.ds(cbase, nch)], cols_buf)
    pltpu.sync_copy(vals_hbm.at[pl.ds(cbase, nch)], vals_buf)

    def quad_body(t4, carry):
        for k in range(NBUF):
            t = t4 * NBUF + k
            xrk = xr[k]
            pltpu.async_copy(xt_hbm.at[rows_buf.at[t]], xrk, gs[k]).wait()

            def scale16(g, c2, _t=t, _xrk=xrk):
                v16 = vals_buf[_t, pl.ds(g * 16, 16)]
                rb = g * 16
                for jj in range(16):
                    v = v16[jj]
                    _xrk[rb + jj, :] = _xrk[rb + jj, :] * v
                return c2
            lax.fori_loop(0, G // 16, scale16, 0)

            pltpu.sync_copy(xrk, yshared.at[cols_buf.at[t]], add=True)
        return carry

    lax.fori_loop(0, nch // NBUF, quad_body, 0)

    plsc.subcore_barrier()

    @pl.when(sid == 0)
    def _writeout():
        pltpu.sync_copy(yshared, ystage_v)
        pltpu.sync_copy(ystage_v, out_hbm.at[cid])


def _sc_partials(xt, rows, cols, vals, nch):
    mesh = plsc.VectorSubcoreMesh(core_axis_name="c", subcore_axis_name="s")
    body = functools.partial(_sc_body, nch=nch)
    return pl.kernel(
        body,
        out_type=jax.ShapeDtypeStruct((NC, N_COMP, BATCH), jnp.float32),
        mesh=mesh,
        scratch_types=[
            pltpu.VMEM((nch, G), jnp.int32),
            pltpu.VMEM((nch, G), jnp.int32),
            pltpu.VMEM((nch, G), jnp.float32),
            pltpu.VMEM((G, BATCH), jnp.float32),
            pltpu.VMEM((G, BATCH), jnp.float32),
            pltpu.VMEM((G, BATCH), jnp.float32),
            pltpu.VMEM((G, BATCH), jnp.float32),
            pltpu.VMEM((N_COMP, BATCH), jnp.float32),
            pltpu.VMEM_SHARED((N_COMP, BATCH), jnp.float32),
            pltpu.SemaphoreType.DMA,
            pltpu.SemaphoreType.DMA,
            pltpu.SemaphoreType.DMA,
            pltpu.SemaphoreType.DMA,
            pltpu.SemaphoreType.DMA,
            pltpu.SemaphoreType.DMA,
            pltpu.SemaphoreType.DMA,
            pltpu.SemaphoreType.DMA,
        ],
        compiler_params=pltpu.CompilerParams(use_tc_tiling_on_sc=False),
    )(xt, rows, cols, vals)


def _tc_body(p_ref, x_ref, b_ref, o_ref):
    ysum = (p_ref[0] + p_ref[1]) * SCALE
    xb = jnp.dot(x_ref[...], b_ref[...], preferred_element_type=jnp.float32)
    y1 = xb.T * SCALE
    pad = jnp.zeros((6, BATCH), jnp.float32)
    o_ref[...] = jnp.concatenate([ysum, y1, pad], axis=0)


def _tc_merge(partials, X, b):
    return pl.pallas_call(
        _tc_body,
        out_shape=jax.ShapeDtypeStruct((N_COMP + N_BG + 6, BATCH), jnp.float32),
    )(partials, X, b)


def kernel(X, A_rows, A_cols, A_vals, b):
    nnz = A_rows.shape[0]
    per_round = NW * G * NBUF
    nnz_pad = ((nnz + per_round - 1) // per_round) * per_round
    pad = nnz_pad - nnz
    rows = jnp.concatenate([A_rows, jnp.zeros((pad,), A_rows.dtype)])
    cols = jnp.concatenate([A_cols, jnp.zeros((pad,), A_cols.dtype)])
    vals = jnp.concatenate([A_vals, jnp.zeros((pad,), A_vals.dtype)])
    rows = rows.reshape(-1, G)
    cols = cols.reshape(-1, G)
    vals = vals.reshape(-1, G)
    xt = X.T
    partials = _sc_partials(xt, rows, cols, vals, nnz_pad // (NW * G))
    out = _tc_merge(partials, X, b)
    return out[:N_COMP + N_BG]


# speedup vs baseline: 31.1923x; 1.6126x over previous
import functools

import jax
import jax.numpy as jnp
from jax import lax
from jax.experimental import pallas as pl
from jax.experimental.pallas import tpu as pltpu
from jax.experimental.pallas import tpu_sc as plsc

N_COMP = 1024
N_BG = 2
BATCH = 16
SCALE = 1.0 / 128.0

NC = 2
NS = 16
NW = NC * NS
G = 128
NBUF = 4


def _sc_body(xt_hbm, rows_hbm, cols_hbm, vals_hbm, out_hbm,
             rows_buf, cols_buf, vals_buf,
             xr0, xr1, xr2, xr3, ystage_v, yshared,
             g0, g1, g2, g3, s0, s1, s2, s3,
             *, nch):
    xr = (xr0, xr1, xr2, xr3)
    gs = (g0, g1, g2, g3)
    ss = (s0, s1, s2, s3)
    cid = lax.axis_index("c")
    sid = lax.axis_index("s")
    wid = sid * NC + cid

    @pl.when(sid == 0)
    def _zero():
        def zrow(i, carry):
            ystage_v[i, :] = jnp.zeros((BATCH,), jnp.float32)
            return carry
        lax.fori_loop(0, N_COMP, zrow, 0)
        pltpu.sync_copy(ystage_v, yshared)

    plsc.subcore_barrier()

    cbase = wid * nch
    pltpu.sync_copy(rows_hbm.at[pl.ds(cbase, nch)], rows_buf)
    pltpu.sync_copy(cols_hbm.at[pl.ds(cbase, nch)], cols_buf)
    pltpu.sync_copy(vals_hbm.at[pl.ds(cbase, nch)], vals_buf)

    pltpu.async_copy(xt_hbm.at[rows_buf.at[0]], xr[0], gs[0])
    pltpu.async_copy(xt_hbm.at[rows_buf.at[1]], xr[1], gs[1])

    def quad_body(t4, carry):
        for k in range(NBUF):
            t = t4 * NBUF + k
            xrk = xr[k]
            pltpu.make_async_copy(xt_hbm.at[rows_buf.at[t]], xrk, gs[k]).wait()
            k2 = (k + 2) % NBUF
            @pl.when(t + 2 < nch)
            def _next_gather():
                pltpu.async_copy(xt_hbm.at[rows_buf.at[t + 2]], xr[k2], gs[k2])

            def scale16(g, c2, _t=t, _xrk=xrk):
                v16 = vals_buf[_t, pl.ds(g * 16, 16)]
                rb = g * 16
                for jj in range(16):
                    v = v16[jj]
                    _xrk[rb + jj, :] = _xrk[rb + jj, :] * v
                return c2
            lax.fori_loop(0, G // 16, scale16, 0)

            pltpu.sync_copy(xrk, yshared.at[cols_buf.at[t]], add=True)
        return carry

    lax.fori_loop(0, nch // NBUF, quad_body, 0)

    plsc.subcore_barrier()

    @pl.when(sid == 0)
    def _writeout():
        pltpu.sync_copy(yshared, ystage_v)
        pltpu.sync_copy(ystage_v, out_hbm.at[cid])


def _sc_partials(xt, rows, cols, vals, nch):
    mesh = plsc.VectorSubcoreMesh(core_axis_name="c", subcore_axis_name="s")
    body = functools.partial(_sc_body, nch=nch)
    return pl.kernel(
        body,
        out_type=jax.ShapeDtypeStruct((NC, N_COMP, BATCH), jnp.float32),
        mesh=mesh,
        scratch_types=[
            pltpu.VMEM((nch, G), jnp.int32),
            pltpu.VMEM((nch, G), jnp.int32),
            pltpu.VMEM((nch, G), jnp.float32),
            pltpu.VMEM((G, BATCH), jnp.float32),
            pltpu.VMEM((G, BATCH), jnp.float32),
            pltpu.VMEM((G, BATCH), jnp.float32),
            pltpu.VMEM((G, BATCH), jnp.float32),
            pltpu.VMEM((N_COMP, BATCH), jnp.float32),
            pltpu.VMEM_SHARED((N_COMP, BATCH), jnp.float32),
            pltpu.SemaphoreType.DMA,
            pltpu.SemaphoreType.DMA,
            pltpu.SemaphoreType.DMA,
            pltpu.SemaphoreType.DMA,
            pltpu.SemaphoreType.DMA,
            pltpu.SemaphoreType.DMA,
            pltpu.SemaphoreType.DMA,
            pltpu.SemaphoreType.DMA,
        ],
        compiler_params=pltpu.CompilerParams(use_tc_tiling_on_sc=False),
    )(xt, rows, cols, vals)


def _tc_body(p_ref, x_ref, b_ref, o_ref):
    ysum = (p_ref[0] + p_ref[1]) * SCALE
    xb = jnp.dot(x_ref[...], b_ref[...], preferred_element_type=jnp.float32)
    y1 = xb.T * SCALE
    pad = jnp.zeros((6, BATCH), jnp.float32)
    o_ref[...] = jnp.concatenate([ysum, y1, pad], axis=0)


def _tc_merge(partials, X, b):
    return pl.pallas_call(
        _tc_body,
        out_shape=jax.ShapeDtypeStruct((N_COMP + N_BG + 6, BATCH), jnp.float32),
    )(partials, X, b)


def kernel(X, A_rows, A_cols, A_vals, b):
    nnz = A_rows.shape[0]
    per_round = NW * G * NBUF
    nnz_pad = ((nnz + per_round - 1) // per_round) * per_round
    pad = nnz_pad - nnz
    rows = jnp.concatenate([A_rows, jnp.zeros((pad,), A_rows.dtype)])
    cols = jnp.concatenate([A_cols, jnp.zeros((pad,), A_cols.dtype)])
    vals = jnp.concatenate([A_vals, jnp.zeros((pad,), A_vals.dtype)])
    rows = rows.reshape(-1, G)
    cols = cols.reshape(-1, G)
    vals = vals.reshape(-1, G)
    xt = X.T
    partials = _sc_partials(xt, rows, cols, vals, nnz_pad // (NW * G))
    out = _tc_merge(partials, X, b)
    return out[:N_COMP + N_BG]


# speedup vs baseline: 31.2486x; 1.0018x over previous
import functools

import jax
import jax.numpy as jnp
from jax import lax
from jax.experimental import pallas as pl
from jax.experimental.pallas import tpu as pltpu
from jax.experimental.pallas import tpu_sc as plsc

N_COMP = 1024
N_BG = 2
BATCH = 16
SCALE = 1.0 / 128.0

NC = 2
NS = 16
NW = NC * NS
G = 128
NBUF = 4


def _sc_body(xt_hbm, rows_hbm, cols_hbm, vals_hbm, out_hbm,
             rows_buf, cols_buf, vals_buf,
             xr0, xr1, xr2, xr3, ystage_v, yshared,
             g0, g1, g2, g3, s0, s1, s2, s3,
             *, nch):
    xr = (xr0, xr1, xr2, xr3)
    gs = (g0, g1, g2, g3)
    ss = (s0, s1, s2, s3)
    cid = lax.axis_index("c")
    sid = lax.axis_index("s")
    wid = sid * NC + cid

    @pl.when(sid == 0)
    def _zero():
        def zrow(i, carry):
            ystage_v[i, :] = jnp.zeros((BATCH,), jnp.float32)
            return carry
        lax.fori_loop(0, N_COMP, zrow, 0)
        pltpu.sync_copy(ystage_v, yshared)

    plsc.subcore_barrier()

    cbase = wid * nch
    pltpu.sync_copy(rows_hbm.at[pl.ds(cbase, nch)], rows_buf)
    pltpu.sync_copy(cols_hbm.at[pl.ds(cbase, nch)], cols_buf)
    pltpu.sync_copy(vals_hbm.at[pl.ds(cbase, nch)], vals_buf)

    pltpu.async_copy(xt_hbm.at[rows_buf.at[0]], xr[0], gs[0])
    pltpu.async_copy(xt_hbm.at[rows_buf.at[1]], xr[1], gs[1])

    def quad_body(t4, carry):
        sdesc = None
        for k in range(NBUF):
            t = t4 * NBUF + k
            xrk = xr[k]
            pltpu.make_async_copy(xt_hbm.at[rows_buf.at[t]], xrk, gs[k]).wait()
            k2 = (k + 2) % NBUF
            @pl.when(t + 2 < nch)
            def _next_gather():
                pltpu.async_copy(xt_hbm.at[rows_buf.at[t + 2]], xr[k2], gs[k2])

            def scale16(g, c2, _t=t, _xrk=xrk):
                v16 = vals_buf[_t, pl.ds(g * 16, 16)]
                rb = g * 16
                for jj in range(16):
                    v = v16[jj]
                    _xrk[rb + jj, :] = _xrk[rb + jj, :] * v
                return c2
            lax.fori_loop(0, G // 16, scale16, 0)

            if sdesc is not None:
                sdesc.wait()
            sdesc = pltpu.async_copy(xrk, yshared.at[cols_buf.at[t]], ss[k])
        sdesc.wait()
        return carry

    lax.fori_loop(0, nch // NBUF, quad_body, 0)

    plsc.subcore_barrier()

    @pl.when(sid == 0)
    def _writeout():
        pltpu.sync_copy(yshared, ystage_v)
        pltpu.sync_copy(ystage_v, out_hbm.at[cid])


def _sc_partials(xt, rows, cols, vals, nch):
    mesh = plsc.VectorSubcoreMesh(core_axis_name="c", subcore_axis_name="s")
    body = functools.partial(_sc_body, nch=nch)
    return pl.kernel(
        body,
        out_type=jax.ShapeDtypeStruct((NC, N_COMP, BATCH), jnp.float32),
        mesh=mesh,
        scratch_types=[
            pltpu.VMEM((nch, G), jnp.int32),
            pltpu.VMEM((nch, G), jnp.int32),
            pltpu.VMEM((nch, G), jnp.float32),
            pltpu.VMEM((G, BATCH), jnp.float32),
            pltpu.VMEM((G, BATCH), jnp.float32),
            pltpu.VMEM((G, BATCH), jnp.float32),
            pltpu.VMEM((G, BATCH), jnp.float32),
            pltpu.VMEM((N_COMP, BATCH), jnp.float32),
            pltpu.VMEM_SHARED((N_COMP, BATCH), jnp.float32),
            pltpu.SemaphoreType.DMA,
            pltpu.SemaphoreType.DMA,
            pltpu.SemaphoreType.DMA,
            pltpu.SemaphoreType.DMA,
            pltpu.SemaphoreType.DMA,
            pltpu.SemaphoreType.DMA,
            pltpu.SemaphoreType.DMA,
            pltpu.SemaphoreType.DMA,
        ],
        compiler_params=pltpu.CompilerParams(use_tc_tiling_on_sc=False),
    )(xt, rows, cols, vals)


def _tc_body(p_ref, x_ref, b_ref, o_ref):
    ysum = (p_ref[0] + p_ref[1]) * SCALE
    xb = jnp.dot(x_ref[...], b_ref[...], preferred_element_type=jnp.float32)
    y1 = xb.T * SCALE
    pad = jnp.zeros((6, BATCH), jnp.float32)
    o_ref[...] = jnp.concatenate([ysum, y1, pad], axis=0)


def _tc_merge(partials, X, b):
    return pl.pallas_call(
        _tc_body,
        out_shape=jax.ShapeDtypeStruct((N_COMP + N_BG + 6, BATCH), jnp.float32),
    )(partials, X, b)


def kernel(X, A_rows, A_cols, A_vals, b):
    nnz = A_rows.shape[0]
    per_round = NW * G * NBUF
    nnz_pad = ((nnz + per_round - 1) // per_round) * per_round
    pad = nnz_pad - nnz
    rows = jnp.concatenate([A_rows, jnp.zeros((pad,), A_rows.dtype)])
    cols = jnp.concatenate([A_cols, jnp.zeros((pad,), A_cols.dtype)])
    vals = jnp.concatenate([A_vals, jnp.zeros((pad,), A_vals.dtype)])
    rows = rows.reshape(-1, G)
    cols = cols.reshape(-1, G)
    vals = vals.reshape(-1, G)
    xt = X.T
    partials = _sc_partials(xt, rows, cols, vals, nnz_pad // (NW * G))
    out = _tc_merge(partials, X, b)
    return out[:N_COMP + N_BG]
